# baseline (speedup 1.0000x reference)
import functools

import jax
import jax.numpy as jnp
from jax import lax
from jax.experimental import pallas as pl
from jax.experimental.pallas import tpu as pltpu
from jax.experimental.pallas import tpu_sc as plsc

N = 10000
E = 160000
D = 256
H = 256
C = 64
G = 64

NC = 2
NS = 16
HH = H // NC

NPAD = 10240

DEG_W = 40
DEG_WINS = E // (NC * NS * DEG_W)
MSG_W = 80
MSG_WINS = E // (NS * MSG_W)

ROWS_PER_SUB = NPAD // NS

_SC_MESH = plsc.VectorSubcoreMesh(core_axis_name="c", subcore_axis_name="s",
                                  num_cores=NC, num_subcores=NS)


def _fill(ref, n, value):
    def body(i, carry):
        ref[pl.ds(i * 16, 16)] = jnp.full((16,), value, ref.dtype)
        return carry
    lax.fori_loop(0, n // 16, body, 0)
    if n % 16:
        ref[pl.ds(n - 16, 16)] = jnp.full((16,), value, ref.dtype)


def _sc_deg_body(dst4_hbm, out_hbm, idx_v, ones_v, zeros_v, acc_sh, sem):
    cid = lax.axis_index("c")
    sid = lax.axis_index("s")
    _fill(ones_v, DEG_W, 1.0)
    _fill(zeros_v, ROWS_PER_SUB, 0.0)
    pltpu.sync_copy(zeros_v, acc_sh.at[pl.ds(sid * ROWS_PER_SUB,
                                             ROWS_PER_SUB)])
    wid = sid * NC + cid
    pltpu.sync_copy(dst4_hbm.at[wid], idx_v)
    plsc.subcore_barrier()

    def fire(wi, carry):
        pltpu.async_copy(ones_v, acc_sh.at[idx_v.at[wi]], sem, add=True)
        return carry
    lax.fori_loop(0, DEG_WINS, fire, 0)

    def drain(wi, carry):
        pltpu.make_async_copy(ones_v, acc_sh.at[idx_v.at[0]], sem).wait()
        return carry
    lax.fori_loop(0, DEG_WINS, drain, 0)
    plsc.subcore_barrier()
    pltpu.sync_copy(
        acc_sh.at[pl.ds(sid * ROWS_PER_SUB, ROWS_PER_SUB)],
        out_hbm.at[pl.ds(cid * NPAD + sid * ROWS_PER_SUB, ROWS_PER_SUB)])


@functools.partial(
    pl.kernel,
    out_type=jax.ShapeDtypeStruct((NC * NPAD,), jnp.float32),
    mesh=_SC_MESH,
    scratch_types=[
        pltpu.VMEM((DEG_WINS, DEG_W), jnp.int32),
        pltpu.VMEM((DEG_W,), jnp.float32),
        pltpu.VMEM((ROWS_PER_SUB,), jnp.float32),
        pltpu.VMEM_SHARED((NPAD,), jnp.float32),
        pltpu.SemaphoreType.DMA,
    ],
)
def _sc_deg(dst4_hbm, out_hbm, idx_v, ones_v, zeros_v, acc_sh, sem):
    _sc_deg_body(dst4_hbm, out_hbm, idx_v, ones_v, zeros_v, acc_sh, sem)


def _sc_msg_body(xls_hbm, src8_hbm, dst8_hbm, out_hbm,
                 sidx0, sidx1, didx_v, rows0, rows1, acc_sh,
                 sem_i0, sem_i1, sem_g0, sem_g1, sem_s0, sem_s1):
    cid = lax.axis_index("c")
    sid = lax.axis_index("s")
    def zrow(i, carry):
        for j in range(HH // 16):
            rows0[i, pl.ds(j * 16, 16)] = jnp.zeros((16,), jnp.float32)
        return carry
    lax.fori_loop(0, MSG_W, zrow, 0)
    for k in range(ROWS_PER_SUB // MSG_W):
        pltpu.sync_copy(
            rows0,
            acc_sh.at[pl.ds(sid * ROWS_PER_SUB + k * MSG_W, MSG_W)])

    pltpu.sync_copy(dst8_hbm.at[sid], didx_v)
    row0 = cid * NPAD

    rows = (rows0, rows1)
    sidx = (sidx0, sidx1)
    sem_i = (sem_i0, sem_i1)
    sem_g = (sem_g0, sem_g1)
    sem_s = (sem_s0, sem_s1)

    base = sid * MSG_WINS * MSG_W

    def load_src(p, w):
        pltpu.async_copy(src8_hbm.at[pl.ds(base + w * MSG_W, MSG_W)],
                         sidx[p], sem_i[p])

    def wait_src(p):
        pltpu.make_async_copy(src8_hbm.at[pl.ds(0, MSG_W)], sidx[p],
                              sem_i[p]).wait()

    def adjust(p):
        for j in range(MSG_W // 16):
            sl = pl.ds(j * 16, 16)
            sidx[p][sl] = sidx[p][sl] + row0

    def start_gather(p):
        pltpu.async_copy(xls_hbm.at[sidx[p]], rows[p], sem_g[p])

    def wait_gather(p):
        pltpu.make_async_copy(xls_hbm.at[sidx[p]], rows[p], sem_g[p]).wait()

    def start_scat(p, w):
        pltpu.async_copy(rows[p], acc_sh.at[didx_v.at[w]], sem_s[p],
                         add=True)

    def wait_scat(p):
        pltpu.make_async_copy(rows[p], acc_sh.at[didx_v.at[0]],
                              sem_s[p]).wait()

    plsc.subcore_barrier()
    load_src(0, 0)
    wait_src(0)
    adjust(0)
    start_gather(0)
    load_src(1, 1)
    wait_gather(0)
    start_scat(0, 0)
    wait_src(1)
    adjust(1)
    start_gather(1)
    load_src(0, 2)

    def pair(g, carry):
        for h in range(2):
            w = 2 * g + 1 + h
            p = (1 + h) % 2
            q = 1 - p
            wait_gather(p)
            start_scat(p, w)
            wait_src(q)
            adjust(q)
            wait_scat(q)
            start_gather(q)
            load_src(p, w + 2)
        return carry
    lax.fori_loop(0, (MSG_WINS - 3) // 2, pair, 0)
    w = MSG_WINS - 2
    wait_gather(1)
    start_scat(1, w)
    wait_src(0)
    adjust(0)
    wait_scat(0)
    start_gather(0)
    wait_gather(0)
    start_scat(0, MSG_WINS - 1)
    wait_scat(1)
    wait_scat(0)
    plsc.subcore_barrier()
    pltpu.sync_copy(
        acc_sh.at[pl.ds(sid * ROWS_PER_SUB, ROWS_PER_SUB)],
        out_hbm.at[pl.ds(cid * NPAD + sid * ROWS_PER_SUB, ROWS_PER_SUB)])


@functools.partial(
    pl.kernel,
    out_type=jax.ShapeDtypeStruct((NC * NPAD, HH), jnp.float32),
    mesh=_SC_MESH,
    scratch_types=[
        pltpu.VMEM((MSG_W,), jnp.int32),
        pltpu.VMEM((MSG_W,), jnp.int32),
        pltpu.VMEM((MSG_WINS, MSG_W), jnp.int32),
        pltpu.VMEM((MSG_W, HH), jnp.float32),
        pltpu.VMEM((MSG_W, HH), jnp.float32),
        pltpu.VMEM_SHARED((NPAD, HH), jnp.float32),
        pltpu.SemaphoreType.DMA,
        pltpu.SemaphoreType.DMA,
        pltpu.SemaphoreType.DMA,
        pltpu.SemaphoreType.DMA,
        pltpu.SemaphoreType.DMA,
        pltpu.SemaphoreType.DMA,
    ],
)
def _sc_msg(xls_hbm, src8_hbm, dst8_hbm, out_hbm,
            sidx0, sidx1, didx_v, rows0, rows1, acc_sh,
            sem_i0, sem_i1, sem_g0, sem_g1, sem_s0, sem_s1):
    _sc_msg_body(xls_hbm, src8_hbm, dst8_hbm, out_hbm,
                 sidx0, sidx1, didx_v, rows0, rows1, acc_sh,
                 sem_i0, sem_i1, sem_g0, sem_g1, sem_s0, sem_s1)


def _row_mask(shape):
    return jnp.where(lax.broadcasted_iota(jnp.int32, shape, 0) < N, 1.0, 0.0)


def _mm_scale_body(x_ref, W_ref, b_ref, degp_ref, dinv_ref, xls_ref):
    deg = degp_ref[0] + degp_ref[1] + 1.0
    dinv = lax.rsqrt(deg)
    @pl.when(pl.program_id(0) == 0)
    def _():
        dinv_ref[...] = dinv
    xl = (jnp.dot(x_ref[...], W_ref[...],
                  preferred_element_type=jnp.float32) + b_ref[...])
    xls_ref[...] = xl * dinv


def _tc_mm_scale(x_pad, W, b, degp):
    return pl.pallas_call(
        _mm_scale_body,
        grid=(NC,),
        in_specs=[
            pl.BlockSpec((NPAD, D), lambda c: (0, 0)),
            pl.BlockSpec((D, HH), lambda c: (0, c)),
            pl.BlockSpec((1, HH), lambda c: (0, c)),
            pl.BlockSpec((NC, NPAD, 1), lambda c: (0, 0, 0)),
        ],
        out_specs=[
            pl.BlockSpec((NPAD, 1), lambda c: (0, 0)),
            pl.BlockSpec((NPAD, HH), lambda c: (c, 0)),
        ],
        out_shape=[
            jax.ShapeDtypeStruct((NPAD, 1), jnp.float32),
            jax.ShapeDtypeStruct((NC * NPAD, HH), jnp.float32),
        ],
    )(x_pad, W, b[None, :], degp)


def _bn_relu(msg_ref, xls_ref, dinv, gamma_ref, beta_ref):
    out = jnp.concatenate([msg_ref[:NPAD] + xls_ref[:NPAD],
                           msg_ref[NPAD:] + xls_ref[NPAD:]], axis=1)
    out = out * dinv
    mask = _row_mask((NPAD, 1))
    om = out * mask
    mu = jnp.sum(om, axis=0, keepdims=True) * (1.0 / N)
    var = jnp.sum(om * om, axis=0, keepdims=True) * (1.0 / N) - mu * mu
    hn = (out - mu) * lax.rsqrt(var + 1e-5) * gamma_ref[...] + beta_ref[...]
    return jnp.maximum(hn, 0.0)


def _mid_body(msg_ref, xls_ref, dinv_ref, gamma_ref, beta_ref, W2_ref,
              b2_ref, xls2_ref):
    dinv = dinv_ref[...]
    h1 = _bn_relu(msg_ref, xls_ref, dinv, gamma_ref, beta_ref)
    xl2 = (jnp.dot(h1, W2_ref[...], preferred_element_type=jnp.float32)
           + b2_ref[...])
    xls2_ref[...] = xl2 * dinv


def _tc_mid(msg1, xls1, dinv, gamma1, beta1, W2, b2):
    return pl.pallas_call(
        _mid_body,
        grid=(NC,),
        in_specs=[
            pl.BlockSpec((NC * NPAD, HH), lambda c: (0, 0)),
            pl.BlockSpec((NC * NPAD, HH), lambda c: (0, 0)),
            pl.BlockSpec((NPAD, 1), lambda c: (0, 0)),
            pl.BlockSpec((1, H), lambda c: (0, 0)),
            pl.BlockSpec((1, H), lambda c: (0, 0)),
            pl.BlockSpec((H, HH), lambda c: (0, c)),
            pl.BlockSpec((1, HH), lambda c: (0, c)),
        ],
        out_specs=pl.BlockSpec((NPAD, HH), lambda c: (c, 0)),
        out_shape=jax.ShapeDtypeStruct((NC * NPAD, HH), jnp.float32),
    )(msg1, xls1, dinv, gamma1[None, :], beta1[None, :], W2, b2[None, :])


def _final_body(msg_ref, xls_ref, dinv_ref, gamma_ref, beta_ref, batch_ref,
                hp_ref, cp_ref, Wih_ref, Whh_ref, b_ref, Wfc_ref, bfc_ref,
                logp_ref, hn_ref, cn_ref):
    dinv = dinv_ref[...]
    h2 = _bn_relu(msg_ref, xls_ref, dinv, gamma_ref, beta_ref)
    seg = lax.broadcasted_iota(jnp.int32, (G, NPAD), 0)
    onehotT = jnp.where(seg == batch_ref[...], 1.0, 0.0)
    psum = jnp.dot(onehotT, h2, preferred_element_type=jnp.float32)
    cnt = jnp.sum(onehotT, axis=1, keepdims=True)
    pooled = psum / jnp.maximum(cnt, 1.0)
    gates = (jnp.dot(pooled, Wih_ref[...], preferred_element_type=jnp.float32)
             + jnp.dot(hp_ref[...], Whh_ref[...],
                       preferred_element_type=jnp.float32) + b_ref[...])
    i = jax.nn.sigmoid(gates[:, 0:H])
    f = jax.nn.sigmoid(gates[:, H:2 * H])
    g = jnp.tanh(gates[:, 2 * H:3 * H])
    o = jax.nn.sigmoid(gates[:, 3 * H:4 * H])
    cn = f * cp_ref[...] + i * g
    hn = o * jnp.tanh(cn)
    logits = (jnp.dot(hn, Wfc_ref[...], preferred_element_type=jnp.float32)
              + bfc_ref[...])
    m = jnp.max(logits, axis=-1, keepdims=True)
    s = logits - m
    lse = jnp.log(jnp.sum(jnp.exp(s), axis=-1, keepdims=True))
    logp_ref[...] = s - lse
    hn_ref[...] = hn
    cn_ref[...] = cn


def _tc_final(msg2, xls2, dinv, gamma2, beta2, batch_pad, hp, cp, WihT, WhhT,
              bsum, Wfc, bfc):
    return pl.pallas_call(
        _final_body,
        out_shape=(
            jax.ShapeDtypeStruct((G, C), jnp.float32),
            jax.ShapeDtypeStruct((G, H), jnp.float32),
            jax.ShapeDtypeStruct((G, H), jnp.float32),
        ),
    )(msg2, xls2, dinv, gamma2[None, :], beta2[None, :], batch_pad[None, :],
      hp, cp, WihT, WhhT, bsum, Wfc, bfc[None, :])


def kernel(x, edge_index, batch, h0, c0, W1, b1, gamma1, beta1, W2, b2,
           gamma2, beta2, W_ih, W_hh, b_ih, b_hh, Wfc, bfc):
    src = edge_index[0]
    dst = edge_index[1]
    dst8 = dst.reshape(NS, MSG_WINS, MSG_W)
    dst4 = dst.reshape(NC * NS, DEG_WINS, DEG_W)
    x_pad = jnp.pad(x, ((0, NPAD - N), (0, 0)))
    batch_pad = jnp.pad(batch, (0, NPAD - N), constant_values=G)
    degp = _sc_deg(dst4).reshape(NC, NPAD, 1)
    dinv, xls1 = _tc_mm_scale(x_pad, W1, b1, degp)
    msg1 = _sc_msg(xls1, src, dst8)
    xls2 = _tc_mid(msg1, xls1, dinv, gamma1, beta1, W2, b2)
    msg2 = _sc_msg(xls2, src, dst8)
    logp, hn, cn = _tc_final(
        msg2, xls2, dinv, gamma2, beta2, batch_pad, h0[0], c0[0],
        W_ih.T, W_hh.T, (b_ih + b_hh)[None, :], Wfc, bfc)
    return (logp, hn[None], cn[None])


# speedup vs baseline: 1.0099x; 1.0099x over previous
import functools

import jax
import jax.numpy as jnp
from jax import lax
from jax.experimental import pallas as pl
from jax.experimental.pallas import tpu as pltpu
from jax.experimental.pallas import tpu_sc as plsc

N = 10000
E = 160000
D = 256
H = 256
C = 64
G = 64

NC = 2
NS = 16
HH = H // NC

NPAD = 10240

DEG_W = 40
DEG_WINS = E // (NC * NS * DEG_W)
MSG_W = 80
MSG_WINS = E // (NS * MSG_W)

ROWS_PER_SUB = NPAD // NS

_SC_MESH = plsc.VectorSubcoreMesh(core_axis_name="c", subcore_axis_name="s",
                                  num_cores=NC, num_subcores=NS)


def _fill(ref, n, value):
    def body(i, carry):
        ref[pl.ds(i * 16, 16)] = jnp.full((16,), value, ref.dtype)
        return carry
    lax.fori_loop(0, n // 16, body, 0)
    if n % 16:
        ref[pl.ds(n - 16, 16)] = jnp.full((16,), value, ref.dtype)


def _sc_deg_body(dst4_hbm, out_hbm, idx_v, ones_v, zeros_v, acc_sh, sem):
    cid = lax.axis_index("c")
    sid = lax.axis_index("s")
    _fill(ones_v, DEG_W, 1.0)
    _fill(zeros_v, ROWS_PER_SUB, 0.0)
    pltpu.sync_copy(zeros_v, acc_sh.at[pl.ds(sid * ROWS_PER_SUB,
                                             ROWS_PER_SUB)])
    wid = sid * NC + cid
    pltpu.sync_copy(dst4_hbm.at[wid], idx_v)
    plsc.subcore_barrier()

    def fire(wi, carry):
        pltpu.async_copy(ones_v, acc_sh.at[idx_v.at[wi]], sem, add=True)
        return carry
    lax.fori_loop(0, DEG_WINS, fire, 0)

    def drain(wi, carry):
        pltpu.make_async_copy(ones_v, acc_sh.at[idx_v.at[0]], sem).wait()
        return carry
    lax.fori_loop(0, DEG_WINS, drain, 0)
    plsc.subcore_barrier()
    pltpu.sync_copy(
        acc_sh.at[pl.ds(sid * ROWS_PER_SUB, ROWS_PER_SUB)],
        out_hbm.at[pl.ds(cid * NPAD + sid * ROWS_PER_SUB, ROWS_PER_SUB)])


@functools.partial(
    pl.kernel,
    out_type=jax.ShapeDtypeStruct((NC * NPAD,), jnp.float32),
    mesh=_SC_MESH,
    scratch_types=[
        pltpu.VMEM((DEG_WINS, DEG_W), jnp.int32),
        pltpu.VMEM((DEG_W,), jnp.float32),
        pltpu.VMEM((ROWS_PER_SUB,), jnp.float32),
        pltpu.VMEM_SHARED((NPAD,), jnp.float32),
        pltpu.SemaphoreType.DMA,
    ],
)
def _sc_deg(dst4_hbm, out_hbm, idx_v, ones_v, zeros_v, acc_sh, sem):
    _sc_deg_body(dst4_hbm, out_hbm, idx_v, ones_v, zeros_v, acc_sh, sem)


def _sc_msg_body(xls_hbm, src8_hbm, dst8_hbm, out_hbm,
                 sidx0, sidx1, didx_v, rows0, rows1, acc_sh,
                 sem_i0, sem_i1, sem_g0, sem_g1, sem_s0, sem_s1):
    cid = lax.axis_index("c")
    sid = lax.axis_index("s")
    def zrow(i, carry):
        for j in range(HH // 16):
            rows0[i, pl.ds(j * 16, 16)] = jnp.zeros((16,), jnp.float32)
        return carry
    lax.fori_loop(0, MSG_W, zrow, 0)
    for k in range(ROWS_PER_SUB // MSG_W):
        pltpu.sync_copy(
            rows0,
            acc_sh.at[pl.ds(sid * ROWS_PER_SUB + k * MSG_W, MSG_W)])

    pltpu.sync_copy(dst8_hbm.at[sid], didx_v)
    row0 = cid * NPAD

    rows = (rows0, rows1)
    sidx = (sidx0, sidx1)
    sem_i = (sem_i0, sem_i1)
    sem_g = (sem_g0, sem_g1)
    sem_s = (sem_s0, sem_s1)

    base = sid * MSG_WINS * MSG_W

    def load_src(p, w):
        pltpu.async_copy(src8_hbm.at[pl.ds(base + w * MSG_W, MSG_W)],
                         sidx[p], sem_i[p])

    def wait_src(p):
        pltpu.make_async_copy(src8_hbm.at[pl.ds(0, MSG_W)], sidx[p],
                              sem_i[p]).wait()

    def adjust(p):
        for j in range(MSG_W // 16):
            sl = pl.ds(j * 16, 16)
            sidx[p][sl] = sidx[p][sl] + row0

    def start_gather(p):
        pltpu.async_copy(xls_hbm.at[sidx[p]], rows[p], sem_g[p])

    def wait_gather(p):
        pltpu.make_async_copy(xls_hbm.at[sidx[p]], rows[p], sem_g[p]).wait()

    def start_scat(p, w):
        pltpu.async_copy(rows[p], acc_sh.at[didx_v.at[w]], sem_s[p],
                         add=True)

    def wait_scat(p):
        pltpu.make_async_copy(rows[p], acc_sh.at[didx_v.at[0]],
                              sem_s[p]).wait()

    plsc.subcore_barrier()
    load_src(0, 0)
    wait_src(0)
    adjust(0)
    start_gather(0)
    load_src(1, 1)
    wait_gather(0)
    start_scat(0, 0)
    wait_src(1)
    adjust(1)
    start_gather(1)
    load_src(0, 2)

    def pair(g, carry):
        for h in range(2):
            w = 2 * g + 1 + h
            p = (1 + h) % 2
            q = 1 - p
            wait_gather(p)
            start_scat(p, w)
            wait_src(q)
            adjust(q)
            wait_scat(q)
            start_gather(q)
            load_src(p, w + 2)
        return carry
    lax.fori_loop(0, (MSG_WINS - 3) // 2, pair, 0)
    w = MSG_WINS - 2
    wait_gather(1)
    start_scat(1, w)
    wait_src(0)
    adjust(0)
    wait_scat(0)
    start_gather(0)
    wait_gather(0)
    start_scat(0, MSG_WINS - 1)
    wait_scat(1)
    wait_scat(0)
    plsc.subcore_barrier()
    pltpu.sync_copy(
        acc_sh.at[pl.ds(sid * ROWS_PER_SUB, ROWS_PER_SUB)],
        out_hbm.at[pl.ds(cid * NPAD + sid * ROWS_PER_SUB, ROWS_PER_SUB)])


@functools.partial(
    pl.kernel,
    out_type=jax.ShapeDtypeStruct((NC * NPAD, HH), jnp.float32),
    mesh=_SC_MESH,
    scratch_types=[
        pltpu.VMEM((MSG_W,), jnp.int32),
        pltpu.VMEM((MSG_W,), jnp.int32),
        pltpu.VMEM((MSG_WINS, MSG_W), jnp.int32),
        pltpu.VMEM((MSG_W, HH), jnp.float32),
        pltpu.VMEM((MSG_W, HH), jnp.float32),
        pltpu.VMEM_SHARED((NPAD, HH), jnp.float32),
        pltpu.SemaphoreType.DMA,
        pltpu.SemaphoreType.DMA,
        pltpu.SemaphoreType.DMA,
        pltpu.SemaphoreType.DMA,
        pltpu.SemaphoreType.DMA,
        pltpu.SemaphoreType.DMA,
    ],
)
def _sc_msg(xls_hbm, src8_hbm, dst8_hbm, out_hbm,
            sidx0, sidx1, didx_v, rows0, rows1, acc_sh,
            sem_i0, sem_i1, sem_g0, sem_g1, sem_s0, sem_s1):
    _sc_msg_body(xls_hbm, src8_hbm, dst8_hbm, out_hbm,
                 sidx0, sidx1, didx_v, rows0, rows1, acc_sh,
                 sem_i0, sem_i1, sem_g0, sem_g1, sem_s0, sem_s1)


def _row_mask(shape):
    return jnp.where(lax.broadcasted_iota(jnp.int32, shape, 0) < N, 1.0, 0.0)


def _mm_scale_body(x_ref, W_ref, b_ref, degp_ref, dinv_ref, xls_ref):
    deg = degp_ref[0] + degp_ref[1] + 1.0
    dinv = lax.rsqrt(deg)
    @pl.when(pl.program_id(0) == 0)
    def _():
        dinv_ref[...] = dinv
    xl = (jnp.dot(x_ref[...], W_ref[...],
                  preferred_element_type=jnp.float32) + b_ref[...])
    xls_ref[pl.ds(0, N)] = xl * dinv[:N]
    xls_ref[pl.ds(N, NPAD - N)] = jnp.zeros((NPAD - N, HH), jnp.float32)


def _tc_mm_scale(x, W, b, degp):
    return pl.pallas_call(
        _mm_scale_body,
        grid=(NC,),
        in_specs=[
            pl.BlockSpec((N, D), lambda c: (0, 0)),
            pl.BlockSpec((D, HH), lambda c: (0, c)),
            pl.BlockSpec((1, HH), lambda c: (0, c)),
            pl.BlockSpec((NC, NPAD, 1), lambda c: (0, 0, 0)),
        ],
        out_specs=[
            pl.BlockSpec((NPAD, 1), lambda c: (0, 0)),
            pl.BlockSpec((NPAD, HH), lambda c: (c, 0)),
        ],
        out_shape=[
            jax.ShapeDtypeStruct((NPAD, 1), jnp.float32),
            jax.ShapeDtypeStruct((NC * NPAD, HH), jnp.float32),
        ],
    )(x, W, b[None, :], degp)


def _bn_relu(msg_ref, xls_ref, dinv, gamma_ref, beta_ref):
    out = jnp.concatenate([msg_ref[:NPAD] + xls_ref[:NPAD],
                           msg_ref[NPAD:] + xls_ref[NPAD:]], axis=1)
    out = out * dinv
    mask = _row_mask((NPAD, 1))
    om = out * mask
    mu = jnp.sum(om, axis=0, keepdims=True) * (1.0 / N)
    var = jnp.sum(om * om, axis=0, keepdims=True) * (1.0 / N) - mu * mu
    hn = (out - mu) * lax.rsqrt(var + 1e-5) * gamma_ref[...] + beta_ref[...]
    return jnp.maximum(hn, 0.0)


def _mid_body(msg_ref, xls_ref, dinv_ref, gamma_ref, beta_ref, W2_ref,
              b2_ref, xls2_ref):
    dinv = dinv_ref[...]
    h1 = _bn_relu(msg_ref, xls_ref, dinv, gamma_ref, beta_ref)
    xl2 = (jnp.dot(h1, W2_ref[...], preferred_element_type=jnp.float32)
           + b2_ref[...])
    xls2_ref[...] = xl2 * dinv


def _tc_mid(msg1, xls1, dinv, gamma1, beta1, W2, b2):
    return pl.pallas_call(
        _mid_body,
        grid=(NC,),
        in_specs=[
            pl.BlockSpec((NC * NPAD, HH), lambda c: (0, 0)),
            pl.BlockSpec((NC * NPAD, HH), lambda c: (0, 0)),
            pl.BlockSpec((NPAD, 1), lambda c: (0, 0)),
            pl.BlockSpec((1, H), lambda c: (0, 0)),
            pl.BlockSpec((1, H), lambda c: (0, 0)),
            pl.BlockSpec((H, HH), lambda c: (0, c)),
            pl.BlockSpec((1, HH), lambda c: (0, c)),
        ],
        out_specs=pl.BlockSpec((NPAD, HH), lambda c: (c, 0)),
        out_shape=jax.ShapeDtypeStruct((NC * NPAD, HH), jnp.float32),
    )(msg1, xls1, dinv, gamma1[None, :], beta1[None, :], W2, b2[None, :])


def _final_body(msg_ref, xls_ref, dinv_ref, gamma_ref, beta_ref, batch_ref,
                hp_ref, cp_ref, Wih_ref, Whh_ref, b_ref, Wfc_ref, bfc_ref,
                logp_ref, hn_ref, cn_ref):
    dinv = dinv_ref[...]
    h2 = _bn_relu(msg_ref, xls_ref, dinv, gamma_ref, beta_ref)
    seg = lax.broadcasted_iota(jnp.int32, (G, NPAD), 0)
    onehotT = jnp.where(seg == batch_ref[...], 1.0, 0.0)
    psum = jnp.dot(onehotT, h2, preferred_element_type=jnp.float32)
    cnt = jnp.sum(onehotT, axis=1, keepdims=True)
    pooled = psum / jnp.maximum(cnt, 1.0)
    gates = (jnp.dot(pooled, Wih_ref[...], preferred_element_type=jnp.float32)
             + jnp.dot(hp_ref[...], Whh_ref[...],
                       preferred_element_type=jnp.float32) + b_ref[...])
    i = jax.nn.sigmoid(gates[:, 0:H])
    f = jax.nn.sigmoid(gates[:, H:2 * H])
    g = jnp.tanh(gates[:, 2 * H:3 * H])
    o = jax.nn.sigmoid(gates[:, 3 * H:4 * H])
    cn = f * cp_ref[...] + i * g
    hn = o * jnp.tanh(cn)
    logits = (jnp.dot(hn, Wfc_ref[...], preferred_element_type=jnp.float32)
              + bfc_ref[...])
    m = jnp.max(logits, axis=-1, keepdims=True)
    s = logits - m
    lse = jnp.log(jnp.sum(jnp.exp(s), axis=-1, keepdims=True))
    logp_ref[...] = s - lse
    hn_ref[...] = hn
    cn_ref[...] = cn


def _tc_final(msg2, xls2, dinv, gamma2, beta2, batch_pad, hp, cp, WihT, WhhT,
              bsum, Wfc, bfc):
    return pl.pallas_call(
        _final_body,
        out_shape=(
            jax.ShapeDtypeStruct((G, C), jnp.float32),
            jax.ShapeDtypeStruct((G, H), jnp.float32),
            jax.ShapeDtypeStruct((G, H), jnp.float32),
        ),
    )(msg2, xls2, dinv, gamma2[None, :], beta2[None, :], batch_pad[None, :],
      hp, cp, WihT, WhhT, bsum, Wfc, bfc[None, :])


def kernel(x, edge_index, batch, h0, c0, W1, b1, gamma1, beta1, W2, b2,
           gamma2, beta2, W_ih, W_hh, b_ih, b_hh, Wfc, bfc):
    src = edge_index[0]
    dst = edge_index[1]
    dst8 = dst.reshape(NS, MSG_WINS, MSG_W)
    dst4 = dst.reshape(NC * NS, DEG_WINS, DEG_W)
    batch_pad = jnp.pad(batch, (0, NPAD - N), constant_values=G)
    degp = _sc_deg(dst4).reshape(NC, NPAD, 1)
    dinv, xls1 = _tc_mm_scale(x, W1, b1, degp)
    msg1 = _sc_msg(xls1, src, dst8)
    xls2 = _tc_mid(msg1, xls1, dinv, gamma1, beta1, W2, b2)
    msg2 = _sc_msg(xls2, src, dst8)
    logp, hn, cn = _tc_final(
        msg2, xls2, dinv, gamma2, beta2, batch_pad, h0[0], c0[0],
        W_ih.T, W_hh.T, (b_ih + b_hh)[None, :], Wfc, bfc)
    return (logp, hn[None], cn[None])
